# Initial kernel scaffold; baseline (speedup 1.0000x reference)
import jax
import jax.numpy as jnp
from jax.experimental import pallas as pl


def kernel(x, edge_index, W1, a1_src, a1_dst, b1, W2, a2_src, a2_dst, b2, Wf1, bf1, Wf2, bf2):
    raise NotImplementedError("write your pallas kernel here")


# speedup vs baseline: 22.0641x; 22.0641x over previous
import functools

import jax
import jax.numpy as jnp
from jax import lax
from jax.experimental import pallas as pl
from jax.experimental.pallas import tpu as pltpu
from jax.experimental.pallas import tpu_sc as plsc

N = 10000
E = 320000
CIN = 128
H1 = 128
COUT = 32
NPB = 100
FC_IN = NPB * COUT
FC1 = 256
OUT_DIM = 64

NC = 2
NS = 16
NW = NC * NS
EPT = E // NW
CHUNK = 80
NCHUNK = EPT // CHUNK
RPT = N // NS


def _make_edge_kernel(D):
    mesh = plsc.VectorSubcoreMesh(core_axis_name="c", subcore_axis_name="s")

    @functools.partial(
        pl.kernel,
        mesh=mesh,
        compiler_params=pltpu.CompilerParams(
            use_tc_tiling_on_sc=False, needs_layout_passes=False),
        out_type=(
            jax.ShapeDtypeStruct((NC, N, D), jnp.float32),
            jax.ShapeDtypeStruct((NC, N, 16), jnp.float32),
        ),
        scratch_types=[
            pltpu.VMEM((N,), jnp.float32),
            pltpu.VMEM((N,), jnp.float32),
            pltpu.VMEM((CHUNK,), jnp.int32),
            pltpu.VMEM((CHUNK,), jnp.int32),
            pltpu.VMEM((CHUNK, D), jnp.float32),
            pltpu.VMEM((CHUNK, 16), jnp.float32),
            pltpu.VMEM((CHUNK,), jnp.float32),
            pltpu.VMEM_SHARED((N, D), jnp.float32),
            pltpu.VMEM_SHARED((N, 16), jnp.float32),
            pltpu.SemaphoreType.DMA,
        ],
    )
    def edge_kernel(h_hbm, asrc_hbm, adst_hbm, src_hbm, dst_hbm, zh_hbm,
                    ze_hbm, outh_hbm, oute_hbm, asrc_v, adst_v, sidx, didx,
                    rows, erows, e_buf, acc_h, acc_e, sem):
        cid = lax.axis_index("c")
        sid = lax.axis_index("s")
        wid = sid * NC + cid

        pltpu.sync_copy(asrc_hbm, asrc_v)
        pltpu.sync_copy(adst_hbm, adst_v)
        base = sid * RPT
        pltpu.sync_copy(zh_hbm, acc_h.at[pl.ds(base, RPT)])
        pltpu.sync_copy(ze_hbm, acc_e.at[pl.ds(base, RPT)])
        plsc.subcore_barrier()

        def chunk_body(c, carry):
            off = c * CHUNK
            pltpu.sync_copy(src_hbm.at[wid, pl.ds(off, CHUNK)], sidx)
            pltpu.sync_copy(dst_hbm.at[wid, pl.ds(off, CHUNK)], didx)
            gather = pltpu.async_copy(h_hbm.at[sidx], rows, sem)
            for j in range(CHUNK // 16):
                s16 = sidx[pl.ds(j * 16, 16)]
                d16 = didx[pl.ds(j * 16, 16)]
                a = plsc.load_gather(asrc_v, [s16]) + plsc.load_gather(
                    adst_v, [d16])
                e_buf[pl.ds(j * 16, 16)] = jnp.exp(jnp.maximum(a, 0.2 * a))
            gather.wait()

            def row_body(r, rcarry):
                ev = plsc.load_gather(e_buf, [jnp.full((16,), r, jnp.int32)])
                for col in range(D // 16):
                    rows[r, pl.ds(col * 16, 16)] = (
                        rows[r, pl.ds(col * 16, 16)] * ev)
                erows[r, pl.ds(0, 16)] = ev
                return rcarry

            lax.fori_loop(0, CHUNK, row_body, 0)
            pltpu.sync_copy(rows, acc_h.at[didx], add=True)
            pltpu.sync_copy(erows, acc_e.at[didx], add=True)
            return carry

        lax.fori_loop(0, NCHUNK, chunk_body, 0)
        plsc.subcore_barrier()
        pltpu.sync_copy(acc_h.at[pl.ds(base, RPT)],
                        outh_hbm.at[cid, pl.ds(base, RPT)])
        pltpu.sync_copy(acc_e.at[pl.ds(base, RPT)],
                        oute_hbm.at[cid, pl.ds(base, RPT)])

    return edge_kernel


_edge_l1 = _make_edge_kernel(H1)
_edge_l2 = _make_edge_kernel(COUT)


def _mm_attn_body(x_ref, w_ref, asw_ref, adw_ref, h_ref, as_ref, ad_ref):
    h = jnp.dot(x_ref[...], w_ref[...], preferred_element_type=jnp.float32)
    h_ref[...] = h
    as_ref[...] = jnp.sum(h * asw_ref[...][None, :], axis=1, keepdims=True)
    ad_ref[...] = jnp.sum(h * adw_ref[...][None, :], axis=1, keepdims=True)


def _combine_mm_body(p0_ref, p1_ref, q0_ref, q1_ref, b_ref, w_ref, asw_ref,
                     adw_ref, h2_ref, as_ref, ad_ref):
    num = p0_ref[...] + p1_ref[...]
    s = q0_ref[:, :1] + q1_ref[:, :1]
    h = num / (s + 1e-16) + b_ref[...][None, :]
    h2 = jnp.dot(h, w_ref[...], preferred_element_type=jnp.float32)
    h2_ref[...] = h2
    as_ref[...] = jnp.sum(h2 * asw_ref[...][None, :], axis=1, keepdims=True)
    ad_ref[...] = jnp.sum(h2 * adw_ref[...][None, :], axis=1, keepdims=True)


def _combine_body(p0_ref, p1_ref, q0_ref, q1_ref, b_ref, h_ref):
    num = p0_ref[...] + p1_ref[...]
    s = q0_ref[:, :1] + q1_ref[:, :1]
    h_ref[...] = num / (s + 1e-16) + b_ref[...][None, :]


def _fc_body(g_ref, wf1_ref, bf1_ref, wf2_ref, bf2_ref, out_ref):
    f = jnp.dot(g_ref[...], wf1_ref[...], preferred_element_type=jnp.float32)
    f = jnp.maximum(f + bf1_ref[...][None, :], 0.0)
    out_ref[...] = (
        jnp.dot(f, wf2_ref[...], preferred_element_type=jnp.float32)
        + bf2_ref[...][None, :])


def kernel(x, edge_index, W1, a1_src, a1_dst, b1, W2, a2_src, a2_dst, b2,
           Wf1, bf1, Wf2, bf2):
    src = edge_index[0].reshape(NW, EPT)
    dst = edge_index[1].reshape(NW, EPT)
    zh1 = jnp.zeros((RPT, H1), jnp.float32)
    zh2 = jnp.zeros((RPT, COUT), jnp.float32)
    ze = jnp.zeros((RPT, 16), jnp.float32)

    h1, as1, ad1 = pl.pallas_call(
        _mm_attn_body,
        out_shape=[
            jax.ShapeDtypeStruct((N, H1), jnp.float32),
            jax.ShapeDtypeStruct((N, 1), jnp.float32),
            jax.ShapeDtypeStruct((N, 1), jnp.float32),
        ],
    )(x, W1, a1_src, a1_dst)

    p1, q1 = _edge_l1(h1, as1.reshape(N), ad1.reshape(N), src, dst, zh1, ze)

    h2, as2, ad2 = pl.pallas_call(
        _combine_mm_body,
        out_shape=[
            jax.ShapeDtypeStruct((N, COUT), jnp.float32),
            jax.ShapeDtypeStruct((N, 1), jnp.float32),
            jax.ShapeDtypeStruct((N, 1), jnp.float32),
        ],
    )(p1[0], p1[1], q1[0], q1[1], b1, W2, a2_src, a2_dst)

    p2, q2 = _edge_l2(h2, as2.reshape(N), ad2.reshape(N), src, dst, zh2, ze)

    hf = pl.pallas_call(
        _combine_body,
        out_shape=jax.ShapeDtypeStruct((N, COUT), jnp.float32),
    )(p2[0], p2[1], q2[0], q2[1], b2)

    g = hf.reshape(N // NPB, NPB * COUT)

    out = pl.pallas_call(
        _fc_body,
        out_shape=jax.ShapeDtypeStruct((N // NPB, OUT_DIM), jnp.float32),
    )(g, Wf1, bf1, Wf2, bf2)

    return out


# speedup vs baseline: 26.9218x; 1.2202x over previous
import functools

import jax
import jax.numpy as jnp
from jax import lax
from jax.experimental import pallas as pl
from jax.experimental.pallas import tpu as pltpu
from jax.experimental.pallas import tpu_sc as plsc

N = 10000
E = 320000
CIN = 128
H1 = 128
COUT = 32
NPB = 100
FC_IN = NPB * COUT
FC1 = 256
OUT_DIM = 64

NC = 2
NS = 16
NW = NC * NS
EPT = E // NW
MCHUNK = 80
NMAIN = EPT // MCHUNK
TAIL = EPT - NMAIN * MCHUNK
RPT = N // NS

_SC_PARAMS = pltpu.CompilerParams(
    use_tc_tiling_on_sc=False, needs_layout_passes=False)


def _make_e_kernel():
    mesh = plsc.VectorSubcoreMesh(core_axis_name="c", subcore_axis_name="s")

    @functools.partial(
        pl.kernel,
        mesh=mesh,
        compiler_params=_SC_PARAMS,
        out_type=jax.ShapeDtypeStruct((NW, EPT), jnp.float32),
        scratch_types=[
            pltpu.VMEM((N,), jnp.float32),
            pltpu.VMEM((N,), jnp.float32),
            pltpu.VMEM((EPT,), jnp.int32),
            pltpu.VMEM((EPT,), jnp.int32),
            pltpu.VMEM((EPT,), jnp.float32),
        ],
    )
    def e_kernel(asrc_hbm, adst_hbm, src_hbm, dst_hbm, e_hbm,
                 asrc_v, adst_v, sidx, didx, e_v):
        cid = lax.axis_index("c")
        sid = lax.axis_index("s")
        wid = sid * NC + cid
        pltpu.sync_copy(asrc_hbm, asrc_v)
        pltpu.sync_copy(adst_hbm, adst_v)
        pltpu.sync_copy(src_hbm.at[wid], sidx)
        pltpu.sync_copy(dst_hbm.at[wid], didx)

        def body(i, carry):
            s16 = sidx[pl.ds(i * 16, 16)]
            d16 = didx[pl.ds(i * 16, 16)]
            a = plsc.load_gather(asrc_v, [s16]) + plsc.load_gather(
                adst_v, [d16])
            e_v[pl.ds(i * 16, 16)] = jnp.exp(jnp.maximum(a, 0.2 * a))
            return carry

        lax.fori_loop(0, EPT // 16, body, 0)
        pltpu.sync_copy(e_v, e_hbm.at[wid])

    return e_kernel


_e_pass = _make_e_kernel()


def _make_edge_kernel(D):
    mesh = plsc.VectorSubcoreMesh(core_axis_name="c", subcore_axis_name="s")

    @functools.partial(
        pl.kernel,
        mesh=mesh,
        compiler_params=_SC_PARAMS,
        out_type=(
            jax.ShapeDtypeStruct((NC, N, D), jnp.float32),
            jax.ShapeDtypeStruct((NC, N, 16), jnp.float32),
        ),
        scratch_types=[
            pltpu.VMEM((MCHUNK, D), jnp.float32),
            pltpu.VMEM((MCHUNK, D), jnp.float32),
            pltpu.VMEM((MCHUNK, 16), jnp.float32),
            pltpu.VMEM((MCHUNK, 16), jnp.float32),
            pltpu.VMEM((MCHUNK,), jnp.int32),
            pltpu.VMEM((MCHUNK,), jnp.int32),
            pltpu.VMEM((MCHUNK,), jnp.int32),
            pltpu.VMEM((MCHUNK,), jnp.int32),
            pltpu.VMEM((MCHUNK,), jnp.float32),
            pltpu.VMEM((MCHUNK,), jnp.float32),
            pltpu.VMEM_SHARED((N, D), jnp.float32),
            pltpu.VMEM_SHARED((N, 16), jnp.float32),
            pltpu.SemaphoreType.DMA,
            pltpu.SemaphoreType.DMA,
            pltpu.SemaphoreType.DMA,
            pltpu.SemaphoreType.DMA,
            pltpu.SemaphoreType.DMA,
            pltpu.SemaphoreType.DMA,
        ],
    )
    def edge_kernel(h_hbm, src_hbm, dst_hbm, e_hbm, zh_hbm, ze_hbm,
                    outh_hbm, oute_hbm, rows0, rows1, er0, er1, sidx0, sidx1,
                    didx0, didx1, eb0, eb1,
                    acc_h, acc_e, si0, si1, sg0, sg1, ss0, ss1):
        cid = lax.axis_index("c")
        sid = lax.axis_index("s")
        wid = sid * NC + cid
        rows_ = (rows0, rows1)
        er_ = (er0, er1)
        sidx_ = (sidx0, sidx1)
        didx_ = (didx0, didx1)
        eb_ = (eb0, eb1)
        si_ = (si0, si1)
        sg_ = (sg0, sg1)
        ss_ = (ss0, ss1)

        base = sid * RPT
        pltpu.sync_copy(zh_hbm, acc_h.at[pl.ds(base, RPT)])
        pltpu.sync_copy(ze_hbm, acc_e.at[pl.ds(base, RPT)])
        pltpu.async_copy(src_hbm.at[wid, pl.ds(0, MCHUNK)], sidx0, si0)
        pltpu.async_copy(dst_hbm.at[wid, pl.ds(0, MCHUNK)], didx0, si0)
        pltpu.async_copy(e_hbm.at[wid, pl.ds(0, MCHUNK)], eb0, si0)
        plsc.subcore_barrier()

        def scale_rows(rows, er, eb, nrows):
            def blk(b, carry):
                for k in range(16):
                    r = b * 16 + k
                    ev = plsc.load_gather(
                        eb, [jnp.full((16,), r, jnp.int32)])
                    for col in range(D // 16):
                        rows[r, pl.ds(col * 16, 16)] = (
                            rows[r, pl.ds(col * 16, 16)] * ev)
                    er[r, pl.ds(0, 16)] = ev
                return carry

            lax.fori_loop(0, nrows // 16, blk, 0)

        def do_chunk(c, p):
            q = 1 - p
            rows, er, sidx, didx, eb = (
                rows_[p], er_[p], sidx_[p], didx_[p], eb_[p])
            off = c * MCHUNK
            pltpu.make_async_copy(
                src_hbm.at[wid, pl.ds(off, MCHUNK)], sidx, si_[p]).wait()
            pltpu.make_async_copy(
                dst_hbm.at[wid, pl.ds(off, MCHUNK)], didx, si_[p]).wait()
            pltpu.make_async_copy(
                e_hbm.at[wid, pl.ds(off, MCHUNK)], eb, si_[p]).wait()
            gather = pltpu.async_copy(h_hbm.at[sidx], rows, sg_[p])

            @pl.when(c + 1 < NMAIN)
            def _():
                noff = (c + 1) * MCHUNK
                pltpu.async_copy(
                    src_hbm.at[wid, pl.ds(noff, MCHUNK)], sidx_[q], si_[q])
                pltpu.async_copy(
                    dst_hbm.at[wid, pl.ds(noff, MCHUNK)], didx_[q], si_[q])
                pltpu.async_copy(
                    e_hbm.at[wid, pl.ds(noff, MCHUNK)], eb_[q], si_[q])

            gather.wait()
            scale_rows(rows, er, eb, MCHUNK)
            pltpu.sync_copy(rows, acc_h.at[didx], add=True)
            pltpu.sync_copy(er, acc_e.at[didx], add=True)

        def pair_body(g, carry):
            do_chunk(2 * g, 0)
            do_chunk(2 * g + 1, 1)
            return carry

        lax.fori_loop(0, NMAIN // 2, pair_body, 0)
        if NMAIN % 2:
            do_chunk(NMAIN - 1, 0)

        plsc.subcore_barrier()
        pltpu.sync_copy(acc_h.at[pl.ds(base, RPT)],
                        outh_hbm.at[cid, pl.ds(base, RPT)])
        pltpu.sync_copy(acc_e.at[pl.ds(base, RPT)],
                        oute_hbm.at[cid, pl.ds(base, RPT)])

    return edge_kernel


_edge_l1 = _make_edge_kernel(H1)
_edge_l2 = _make_edge_kernel(COUT)


def _mm_attn_body(x_ref, w_ref, asw_ref, adw_ref, h_ref, as_ref, ad_ref):
    h = jnp.dot(x_ref[...], w_ref[...], preferred_element_type=jnp.float32)
    h_ref[...] = h
    as_ref[...] = jnp.sum(h * asw_ref[...][None, :], axis=1, keepdims=True)
    ad_ref[...] = jnp.sum(h * adw_ref[...][None, :], axis=1, keepdims=True)


def _combine_mm_body(p0_ref, p1_ref, q0_ref, q1_ref, b_ref, w_ref, asw_ref,
                     adw_ref, h2_ref, as_ref, ad_ref):
    num = p0_ref[...] + p1_ref[...]
    s = q0_ref[:, :1] + q1_ref[:, :1]
    h = num / (s + 1e-16) + b_ref[...][None, :]
    h2 = jnp.dot(h, w_ref[...], preferred_element_type=jnp.float32)
    h2_ref[...] = h2
    as_ref[...] = jnp.sum(h2 * asw_ref[...][None, :], axis=1, keepdims=True)
    ad_ref[...] = jnp.sum(h2 * adw_ref[...][None, :], axis=1, keepdims=True)


def _combine_body(p0_ref, p1_ref, q0_ref, q1_ref, b_ref, h_ref):
    num = p0_ref[...] + p1_ref[...]
    s = q0_ref[:, :1] + q1_ref[:, :1]
    h_ref[...] = num / (s + 1e-16) + b_ref[...][None, :]


def _fc_body(g_ref, wf1_ref, bf1_ref, wf2_ref, bf2_ref, out_ref):
    f = jnp.dot(g_ref[...], wf1_ref[...], preferred_element_type=jnp.float32)
    f = jnp.maximum(f + bf1_ref[...][None, :], 0.0)
    out_ref[...] = (
        jnp.dot(f, wf2_ref[...], preferred_element_type=jnp.float32)
        + bf2_ref[...][None, :])


def kernel(x, edge_index, W1, a1_src, a1_dst, b1, W2, a2_src, a2_dst, b2,
           Wf1, bf1, Wf2, bf2):
    src = edge_index[0].reshape(NW, EPT)
    dst = edge_index[1].reshape(NW, EPT)
    zh1 = jnp.zeros((RPT, H1), jnp.float32)
    zh2 = jnp.zeros((RPT, COUT), jnp.float32)
    ze = jnp.zeros((RPT, 16), jnp.float32)

    h1, as1, ad1 = pl.pallas_call(
        _mm_attn_body,
        out_shape=[
            jax.ShapeDtypeStruct((N, H1), jnp.float32),
            jax.ShapeDtypeStruct((N, 1), jnp.float32),
            jax.ShapeDtypeStruct((N, 1), jnp.float32),
        ],
    )(x, W1, a1_src, a1_dst)

    e1 = _e_pass(as1.reshape(N), ad1.reshape(N), src, dst)
    p1, q1 = _edge_l1(h1, src, dst, e1, zh1, ze)

    h2, as2, ad2 = pl.pallas_call(
        _combine_mm_body,
        out_shape=[
            jax.ShapeDtypeStruct((N, COUT), jnp.float32),
            jax.ShapeDtypeStruct((N, 1), jnp.float32),
            jax.ShapeDtypeStruct((N, 1), jnp.float32),
        ],
    )(p1[0], p1[1], q1[0], q1[1], b1, W2, a2_src, a2_dst)

    e2 = _e_pass(as2.reshape(N), ad2.reshape(N), src, dst)
    p2, q2 = _edge_l2(h2, src, dst, e2, zh2, ze)

    hf = pl.pallas_call(
        _combine_body,
        out_shape=jax.ShapeDtypeStruct((N, COUT), jnp.float32),
    )(p2[0], p2[1], q2[0], q2[1], b2)

    g = hf.reshape(N // NPB, NPB * COUT)

    out = pl.pallas_call(
        _fc_body,
        out_shape=jax.ShapeDtypeStruct((N // NPB, OUT_DIM), jnp.float32),
    )(g, Wf1, bf1, Wf2, bf2)

    return out


# speedup vs baseline: 35.2537x; 1.3095x over previous
import functools

import jax
import jax.numpy as jnp
from jax import lax
from jax.experimental import pallas as pl
from jax.experimental.pallas import tpu as pltpu
from jax.experimental.pallas import tpu_sc as plsc

N = 10000
E = 320000
CIN = 128
H1 = 128
COUT = 32
NPB = 100
FC_IN = NPB * COUT
FC1 = 256
OUT_DIM = 64

NC = 2
NS = 16
NW = NC * NS
EPT = E // NW
MCHUNK = 80
NMAIN = EPT // MCHUNK
TAIL = EPT - NMAIN * MCHUNK
RPT = N // NS

_SC_PARAMS = pltpu.CompilerParams(
    use_tc_tiling_on_sc=False, needs_layout_passes=False)


def _make_e_kernel():
    mesh = plsc.VectorSubcoreMesh(core_axis_name="c", subcore_axis_name="s")

    @functools.partial(
        pl.kernel,
        mesh=mesh,
        compiler_params=_SC_PARAMS,
        out_type=jax.ShapeDtypeStruct((NW, EPT), jnp.float32),
        scratch_types=[
            pltpu.VMEM((N,), jnp.float32),
            pltpu.VMEM((N,), jnp.float32),
            pltpu.VMEM((EPT,), jnp.int32),
            pltpu.VMEM((EPT,), jnp.int32),
            pltpu.VMEM((EPT,), jnp.float32),
        ],
    )
    def e_kernel(asrc_hbm, adst_hbm, src_hbm, dst_hbm, e_hbm,
                 asrc_v, adst_v, sidx, didx, e_v):
        cid = lax.axis_index("c")
        sid = lax.axis_index("s")
        wid = sid * NC + cid
        pltpu.sync_copy(asrc_hbm, asrc_v)
        pltpu.sync_copy(adst_hbm, adst_v)
        pltpu.sync_copy(src_hbm.at[wid], sidx)
        pltpu.sync_copy(dst_hbm.at[wid], didx)

        def body(i, carry):
            s16 = sidx[pl.ds(i * 16, 16)]
            d16 = didx[pl.ds(i * 16, 16)]
            a = plsc.load_gather(asrc_v, [s16]) + plsc.load_gather(
                adst_v, [d16])
            e_v[pl.ds(i * 16, 16)] = jnp.exp(jnp.maximum(a, 0.2 * a))
            return carry

        lax.fori_loop(0, EPT // 16, body, 0)
        pltpu.sync_copy(e_v, e_hbm.at[wid])

    return e_kernel


_e_pass = _make_e_kernel()


def _make_edge_kernel(D):
    mesh = plsc.VectorSubcoreMesh(core_axis_name="c", subcore_axis_name="s")

    @functools.partial(
        pl.kernel,
        mesh=mesh,
        compiler_params=_SC_PARAMS,
        out_type=(
            jax.ShapeDtypeStruct((NC, N, D), jnp.float32),
            jax.ShapeDtypeStruct((NC, N, 16), jnp.float32),
        ),
        scratch_types=(
            [pltpu.VMEM((MCHUNK, D), jnp.float32)] * 3
            + [pltpu.VMEM((MCHUNK, 16), jnp.float32)] * 3
            + [pltpu.VMEM((MCHUNK,), jnp.int32)] * 3
            + [pltpu.VMEM((MCHUNK,), jnp.int32)] * 3
            + [pltpu.VMEM((MCHUNK,), jnp.float32)] * 3
            + [
                pltpu.VMEM_SHARED((N, D), jnp.float32),
                pltpu.VMEM_SHARED((N, 16), jnp.float32),
            ]
            + [pltpu.SemaphoreType.DMA] * 12
        ),
    )
    def edge_kernel(h_hbm, src_hbm, dst_hbm, e_hbm, zh_hbm, ze_hbm,
                    outh_hbm, oute_hbm,
                    rowsA, rowsB, rowsC, erA, erB, erC, sidxA, sidxB, sidxC,
                    didxA, didxB, didxC, ebA, ebB, ebC, acc_h, acc_e,
                    siA, siB, siC, sdA, sdB, sdC, sgA, sgB, sgC,
                    ssA, ssB, ssC):
        cid = lax.axis_index("c")
        sid = lax.axis_index("s")
        wid = sid * NC + cid
        rows_ = (rowsA, rowsB, rowsC)
        er_ = (erA, erB, erC)
        sidx_ = (sidxA, sidxB, sidxC)
        didx_ = (didxA, didxB, didxC)
        eb_ = (ebA, ebB, ebC)
        si_ = (siA, siB, siC)
        sd_ = (sdA, sdB, sdC)
        sg_ = (sgA, sgB, sgC)
        ss_ = (ssA, ssB, ssC)

        base = sid * RPT
        pltpu.sync_copy(zh_hbm, acc_h.at[pl.ds(base, RPT)])
        pltpu.sync_copy(ze_hbm, acc_e.at[pl.ds(base, RPT)])
        for k in range(3):
            off0 = k * MCHUNK
            pltpu.async_copy(
                src_hbm.at[wid, pl.ds(off0, MCHUNK)], sidx_[k], si_[k])
            pltpu.async_copy(
                dst_hbm.at[wid, pl.ds(off0, MCHUNK)], didx_[k], sd_[k])
            pltpu.async_copy(
                e_hbm.at[wid, pl.ds(off0, MCHUNK)], eb_[k], sd_[k])
        plsc.subcore_barrier()

        def scale_rows(rows, er, eb, nrows):
            def blk(b, carry):
                for k in range(16):
                    r = b * 16 + k
                    ev = plsc.load_gather(
                        eb, [jnp.full((16,), r, jnp.int32)])
                    for col in range(D // 16):
                        rows[r, pl.ds(col * 16, 16)] = (
                            rows[r, pl.ds(col * 16, 16)] * ev)
                    er[r, pl.ds(0, 16)] = ev
                return carry

            lax.fori_loop(0, nrows // 16, blk, 0)

        NBODY = NMAIN // 3
        LEFT = NMAIN - NBODY * 3

        def body(g, carry):
            c0 = 3 * g
            gathers = []
            for k in range(3):
                off = (c0 + k) * MCHUNK
                pltpu.make_async_copy(
                    src_hbm.at[wid, pl.ds(off, MCHUNK)],
                    sidx_[k], si_[k]).wait()
                gathers.append(
                    pltpu.async_copy(h_hbm.at[sidx_[k]], rows_[k], sg_[k]))
            scats = []
            for k in range(3):
                c = c0 + k
                gathers[k].wait()

                @pl.when(c + 3 < NMAIN)
                def _(k=k, c=c):
                    pltpu.async_copy(
                        src_hbm.at[wid, pl.ds((c + 3) * MCHUNK, MCHUNK)],
                        sidx_[k], si_[k])

                off = c * MCHUNK
                pltpu.make_async_copy(
                    dst_hbm.at[wid, pl.ds(off, MCHUNK)],
                    didx_[k], sd_[k]).wait()
                pltpu.make_async_copy(
                    e_hbm.at[wid, pl.ds(off, MCHUNK)], eb_[k], sd_[k]).wait()
                scale_rows(rows_[k], er_[k], eb_[k], MCHUNK)
                scats.append((
                    pltpu.async_copy(
                        rows_[k], acc_h.at[didx_[k]], ss_[k], add=True),
                    pltpu.async_copy(
                        er_[k], acc_e.at[didx_[k]], ss_[k], add=True),
                ))
            for k in range(3):
                c = c0 + k
                scats[k][0].wait()
                scats[k][1].wait()

                @pl.when(c + 3 < NMAIN)
                def _(k=k, c=c):
                    noff = (c + 3) * MCHUNK
                    pltpu.async_copy(
                        dst_hbm.at[wid, pl.ds(noff, MCHUNK)],
                        didx_[k], sd_[k])
                    pltpu.async_copy(
                        e_hbm.at[wid, pl.ds(noff, MCHUNK)], eb_[k], sd_[k])
            return carry

        lax.fori_loop(0, NBODY, body, 0)
        for k in range(LEFT):
            c = NBODY * 3 + k
            off = c * MCHUNK
            pltpu.make_async_copy(
                src_hbm.at[wid, pl.ds(off, MCHUNK)], sidx_[k], si_[k]).wait()
            pltpu.async_copy(h_hbm.at[sidx_[k]], rows_[k], sg_[k]).wait()
            pltpu.make_async_copy(
                dst_hbm.at[wid, pl.ds(off, MCHUNK)], didx_[k], sd_[k]).wait()
            pltpu.make_async_copy(
                e_hbm.at[wid, pl.ds(off, MCHUNK)], eb_[k], sd_[k]).wait()
            scale_rows(rows_[k], er_[k], eb_[k], MCHUNK)
            pltpu.sync_copy(rows_[k], acc_h.at[didx_[k]], add=True)
            pltpu.sync_copy(er_[k], acc_e.at[didx_[k]], add=True)

        plsc.subcore_barrier()
        pltpu.sync_copy(acc_h.at[pl.ds(base, RPT)],
                        outh_hbm.at[cid, pl.ds(base, RPT)])
        pltpu.sync_copy(acc_e.at[pl.ds(base, RPT)],
                        oute_hbm.at[cid, pl.ds(base, RPT)])

    return edge_kernel


_edge_l1 = _make_edge_kernel(H1)
_edge_l2 = _make_edge_kernel(COUT)


def _mm_attn_body(x_ref, w_ref, asw_ref, adw_ref, h_ref, as_ref, ad_ref):
    h = jnp.dot(x_ref[...], w_ref[...], preferred_element_type=jnp.float32)
    h_ref[...] = h
    as_ref[...] = jnp.sum(h * asw_ref[...][None, :], axis=1, keepdims=True)
    ad_ref[...] = jnp.sum(h * adw_ref[...][None, :], axis=1, keepdims=True)


def _combine_mm_body(p0_ref, p1_ref, q0_ref, q1_ref, b_ref, w_ref, asw_ref,
                     adw_ref, h2_ref, as_ref, ad_ref):
    num = p0_ref[...] + p1_ref[...]
    s = q0_ref[:, :1] + q1_ref[:, :1]
    h = num / (s + 1e-16) + b_ref[...][None, :]
    h2 = jnp.dot(h, w_ref[...], preferred_element_type=jnp.float32)
    h2_ref[...] = h2
    as_ref[...] = jnp.sum(h2 * asw_ref[...][None, :], axis=1, keepdims=True)
    ad_ref[...] = jnp.sum(h2 * adw_ref[...][None, :], axis=1, keepdims=True)


def _combine_body(p0_ref, p1_ref, q0_ref, q1_ref, b_ref, h_ref):
    num = p0_ref[...] + p1_ref[...]
    s = q0_ref[:, :1] + q1_ref[:, :1]
    h_ref[...] = num / (s + 1e-16) + b_ref[...][None, :]


def _fc_body(g_ref, wf1_ref, bf1_ref, wf2_ref, bf2_ref, out_ref):
    f = jnp.dot(g_ref[...], wf1_ref[...], preferred_element_type=jnp.float32)
    f = jnp.maximum(f + bf1_ref[...][None, :], 0.0)
    out_ref[...] = (
        jnp.dot(f, wf2_ref[...], preferred_element_type=jnp.float32)
        + bf2_ref[...][None, :])


def kernel(x, edge_index, W1, a1_src, a1_dst, b1, W2, a2_src, a2_dst, b2,
           Wf1, bf1, Wf2, bf2):
    src = edge_index[0].reshape(NW, EPT)
    dst = edge_index[1].reshape(NW, EPT)
    zh1 = jnp.zeros((RPT, H1), jnp.float32)
    zh2 = jnp.zeros((RPT, COUT), jnp.float32)
    ze = jnp.zeros((RPT, 16), jnp.float32)

    h1, as1, ad1 = pl.pallas_call(
        _mm_attn_body,
        out_shape=[
            jax.ShapeDtypeStruct((N, H1), jnp.float32),
            jax.ShapeDtypeStruct((N, 1), jnp.float32),
            jax.ShapeDtypeStruct((N, 1), jnp.float32),
        ],
    )(x, W1, a1_src, a1_dst)

    e1 = _e_pass(as1.reshape(N), ad1.reshape(N), src, dst)
    p1, q1 = _edge_l1(h1, src, dst, e1, zh1, ze)

    h2, as2, ad2 = pl.pallas_call(
        _combine_mm_body,
        out_shape=[
            jax.ShapeDtypeStruct((N, COUT), jnp.float32),
            jax.ShapeDtypeStruct((N, 1), jnp.float32),
            jax.ShapeDtypeStruct((N, 1), jnp.float32),
        ],
    )(p1[0], p1[1], q1[0], q1[1], b1, W2, a2_src, a2_dst)

    e2 = _e_pass(as2.reshape(N), ad2.reshape(N), src, dst)
    p2, q2 = _edge_l2(h2, src, dst, e2, zh2, ze)

    hf = pl.pallas_call(
        _combine_body,
        out_shape=jax.ShapeDtypeStruct((N, COUT), jnp.float32),
    )(p2[0], p2[1], q2[0], q2[1], b2)

    g = hf.reshape(N // NPB, NPB * COUT)

    out = pl.pallas_call(
        _fc_body,
        out_shape=jax.ShapeDtypeStruct((N // NPB, OUT_DIM), jnp.float32),
    )(g, Wf1, bf1, Wf2, bf2)

    return out


# speedup vs baseline: 37.2961x; 1.0579x over previous
import functools

import jax
import jax.numpy as jnp
from jax import lax
from jax.experimental import pallas as pl
from jax.experimental.pallas import tpu as pltpu
from jax.experimental.pallas import tpu_sc as plsc

N = 10000
E = 320000
CIN = 128
H1 = 128
COUT = 32
NPB = 100
FC_IN = NPB * COUT
FC1 = 256
OUT_DIM = 64

NC = 2
NS = 16
NW = NC * NS
EPT = E // NW
MCHUNK = 80
NMAIN = EPT // MCHUNK
TAIL = EPT - NMAIN * MCHUNK
RPT = N // NS

_SC_PARAMS = pltpu.CompilerParams(
    use_tc_tiling_on_sc=False, needs_layout_passes=False)


def _make_e_kernel():
    mesh = plsc.VectorSubcoreMesh(core_axis_name="c", subcore_axis_name="s")

    @functools.partial(
        pl.kernel,
        mesh=mesh,
        compiler_params=_SC_PARAMS,
        out_type=jax.ShapeDtypeStruct((NW, EPT), jnp.float32),
        scratch_types=[
            pltpu.VMEM((N,), jnp.float32),
            pltpu.VMEM((N,), jnp.float32),
            pltpu.VMEM((EPT,), jnp.int32),
            pltpu.VMEM((EPT,), jnp.int32),
            pltpu.VMEM((EPT,), jnp.float32),
        ],
    )
    def e_kernel(asrc_hbm, adst_hbm, src_hbm, dst_hbm, e_hbm,
                 asrc_v, adst_v, sidx, didx, e_v):
        cid = lax.axis_index("c")
        sid = lax.axis_index("s")
        wid = sid * NC + cid
        pltpu.sync_copy(asrc_hbm, asrc_v)
        pltpu.sync_copy(adst_hbm, adst_v)
        pltpu.sync_copy(src_hbm.at[wid], sidx)
        pltpu.sync_copy(dst_hbm.at[wid], didx)

        def body(i, carry):
            s16 = sidx[pl.ds(i * 16, 16)]
            d16 = didx[pl.ds(i * 16, 16)]
            a = plsc.load_gather(asrc_v, [s16]) + plsc.load_gather(
                adst_v, [d16])
            e_v[pl.ds(i * 16, 16)] = jnp.exp(jnp.maximum(a, 0.2 * a))
            return carry

        lax.fori_loop(0, EPT // 16, body, 0)
        pltpu.sync_copy(e_v, e_hbm.at[wid])

    return e_kernel


_e_pass = _make_e_kernel()


def _make_edge_kernel(D):
    mesh = plsc.VectorSubcoreMesh(core_axis_name="c", subcore_axis_name="s")

    @functools.partial(
        pl.kernel,
        mesh=mesh,
        compiler_params=_SC_PARAMS,
        out_type=(
            jax.ShapeDtypeStruct((NC, N, D), jnp.float32),
            jax.ShapeDtypeStruct((NC, N, 16), jnp.float32),
        ),
        scratch_types=(
            [pltpu.VMEM((MCHUNK, D), jnp.float32)] * 3
            + [pltpu.VMEM((MCHUNK, 16), jnp.float32)] * 3
            + [pltpu.VMEM((MCHUNK,), jnp.int32)] * 3
            + [pltpu.VMEM((MCHUNK,), jnp.int32)] * 3
            + [pltpu.VMEM((MCHUNK,), jnp.float32)] * 3
            + [
                pltpu.VMEM_SHARED((N, D), jnp.float32),
                pltpu.VMEM_SHARED((N, 16), jnp.float32),
            ]
            + [pltpu.SemaphoreType.DMA] * 12
        ),
    )
    def edge_kernel(h_hbm, src_hbm, dst_hbm, e_hbm, zh_hbm, ze_hbm,
                    outh_hbm, oute_hbm,
                    rowsA, rowsB, rowsC, erA, erB, erC, sidxA, sidxB, sidxC,
                    didxA, didxB, didxC, ebA, ebB, ebC, acc_h, acc_e,
                    siA, siB, siC, sdA, sdB, sdC, sgA, sgB, sgC,
                    ssA, ssB, ssC):
        cid = lax.axis_index("c")
        sid = lax.axis_index("s")
        wid = sid * NC + cid
        rows_ = (rowsA, rowsB, rowsC)
        er_ = (erA, erB, erC)
        sidx_ = (sidxA, sidxB, sidxC)
        didx_ = (didxA, didxB, didxC)
        eb_ = (ebA, ebB, ebC)
        si_ = (siA, siB, siC)
        sd_ = (sdA, sdB, sdC)
        sg_ = (sgA, sgB, sgC)
        ss_ = (ssA, ssB, ssC)

        base = sid * RPT
        pltpu.sync_copy(zh_hbm, acc_h.at[pl.ds(base, RPT)])
        pltpu.sync_copy(ze_hbm, acc_e.at[pl.ds(base, RPT)])
        for k in range(3):
            off0 = k * MCHUNK
            pltpu.async_copy(
                src_hbm.at[wid, pl.ds(off0, MCHUNK)], sidx_[k], si_[k])
            pltpu.async_copy(
                dst_hbm.at[wid, pl.ds(off0, MCHUNK)], didx_[k], sd_[k])
            pltpu.async_copy(
                e_hbm.at[wid, pl.ds(off0, MCHUNK)], eb_[k], sd_[k])
        plsc.subcore_barrier()

        def scale_rows(rows, er, eb, nrows):
            def blk(b, carry):
                for k in range(16):
                    r = b * 16 + k
                    ev = plsc.load_gather(
                        eb, [jnp.full((16,), r, jnp.int32)])
                    for col in range(D // 16):
                        rows[r, pl.ds(col * 16, 16)] = (
                            rows[r, pl.ds(col * 16, 16)] * ev)
                    er[r, pl.ds(0, 16)] = ev
                return carry

            lax.fori_loop(0, nrows // 16, blk, 0)

        NBODY = NMAIN // 3
        LEFT = NMAIN - NBODY * 3

        def body(g, carry):
            c0 = 3 * g
            gathers = []
            for k in range(3):
                off = (c0 + k) * MCHUNK
                pltpu.make_async_copy(
                    src_hbm.at[wid, pl.ds(off, MCHUNK)],
                    sidx_[k], si_[k]).wait()
                gathers.append(
                    pltpu.async_copy(h_hbm.at[sidx_[k]], rows_[k], sg_[k]))
            scats = []
            for k in range(3):
                c = c0 + k
                gathers[k].wait()

                @pl.when(c + 3 < NMAIN)
                def _(k=k, c=c):
                    pltpu.async_copy(
                        src_hbm.at[wid, pl.ds((c + 3) * MCHUNK, MCHUNK)],
                        sidx_[k], si_[k])

                off = c * MCHUNK
                pltpu.make_async_copy(
                    dst_hbm.at[wid, pl.ds(off, MCHUNK)],
                    didx_[k], sd_[k]).wait()
                pltpu.make_async_copy(
                    e_hbm.at[wid, pl.ds(off, MCHUNK)], eb_[k], sd_[k]).wait()
                scale_rows(rows_[k], er_[k], eb_[k], MCHUNK)
                scats.append((
                    pltpu.async_copy(
                        rows_[k], acc_h.at[didx_[k]], ss_[k], add=True),
                    pltpu.async_copy(
                        er_[k], acc_e.at[didx_[k]], ss_[k], add=True),
                ))
            for k in range(3):
                c = c0 + k
                scats[k][0].wait()
                scats[k][1].wait()

                @pl.when(c + 3 < NMAIN)
                def _(k=k, c=c):
                    noff = (c + 3) * MCHUNK
                    pltpu.async_copy(
                        dst_hbm.at[wid, pl.ds(noff, MCHUNK)],
                        didx_[k], sd_[k])
                    pltpu.async_copy(
                        e_hbm.at[wid, pl.ds(noff, MCHUNK)], eb_[k], sd_[k])
            return carry

        lax.fori_loop(0, NBODY, body, 0)
        for k in range(LEFT):
            c = NBODY * 3 + k
            off = c * MCHUNK
            pltpu.make_async_copy(
                src_hbm.at[wid, pl.ds(off, MCHUNK)], sidx_[k], si_[k]).wait()
            pltpu.async_copy(h_hbm.at[sidx_[k]], rows_[k], sg_[k]).wait()
            pltpu.make_async_copy(
                dst_hbm.at[wid, pl.ds(off, MCHUNK)], didx_[k], sd_[k]).wait()
            pltpu.make_async_copy(
                e_hbm.at[wid, pl.ds(off, MCHUNK)], eb_[k], sd_[k]).wait()
            scale_rows(rows_[k], er_[k], eb_[k], MCHUNK)
            pltpu.sync_copy(rows_[k], acc_h.at[didx_[k]], add=True)
            pltpu.sync_copy(er_[k], acc_e.at[didx_[k]], add=True)

        plsc.subcore_barrier()
        pltpu.sync_copy(acc_h.at[pl.ds(base, RPT)],
                        outh_hbm.at[cid, pl.ds(base, RPT)])
        pltpu.sync_copy(acc_e.at[pl.ds(base, RPT)],
                        oute_hbm.at[cid, pl.ds(base, RPT)])

    return edge_kernel


_edge_l1 = _make_edge_kernel(H1)
_edge_l2 = _make_edge_kernel(COUT)


def _mm_attn_body(x_ref, w_ref, asw_ref, adw_ref, h_ref, as_ref, ad_ref):
    h = jnp.dot(x_ref[...], w_ref[...], preferred_element_type=jnp.float32)
    h_ref[...] = h
    as_ref[...] = jnp.sum(h * asw_ref[...][None, :], axis=1, keepdims=True)
    ad_ref[...] = jnp.sum(h * adw_ref[...][None, :], axis=1, keepdims=True)


def _combine_mm_body(p_ref, q_ref, b_ref, w_ref, asw_ref,
                     adw_ref, h2_ref, as_ref, ad_ref):
    num = p_ref[0] + p_ref[1]
    s = q_ref[0, :, :1] + q_ref[1, :, :1]
    h = num / (s + 1e-16) + b_ref[...][None, :]
    h2 = jnp.dot(h, w_ref[...], preferred_element_type=jnp.float32)
    h2_ref[...] = h2
    as_ref[...] = jnp.sum(h2 * asw_ref[...][None, :], axis=1, keepdims=True)
    ad_ref[...] = jnp.sum(h2 * adw_ref[...][None, :], axis=1, keepdims=True)


def _combine_body(p_ref, q_ref, b_ref, h_ref):
    num = p_ref[0] + p_ref[1]
    s = q_ref[0, :, :1] + q_ref[1, :, :1]
    h_ref[...] = num / (s + 1e-16) + b_ref[...][None, :]


def _fc_body(g_ref, wf1_ref, bf1_ref, wf2_ref, bf2_ref, out_ref):
    f = jnp.dot(g_ref[...], wf1_ref[...], preferred_element_type=jnp.float32)
    f = jnp.maximum(f + bf1_ref[...][None, :], 0.0)
    out_ref[...] = (
        jnp.dot(f, wf2_ref[...], preferred_element_type=jnp.float32)
        + bf2_ref[...][None, :])


def kernel(x, edge_index, W1, a1_src, a1_dst, b1, W2, a2_src, a2_dst, b2,
           Wf1, bf1, Wf2, bf2):
    src = edge_index[0].reshape(NW, EPT)
    dst = edge_index[1].reshape(NW, EPT)
    zh1 = jnp.zeros((RPT, H1), jnp.float32)
    zh2 = jnp.zeros((RPT, COUT), jnp.float32)
    ze = jnp.zeros((RPT, 16), jnp.float32)

    h1, as1, ad1 = pl.pallas_call(
        _mm_attn_body,
        out_shape=[
            jax.ShapeDtypeStruct((N, H1), jnp.float32),
            jax.ShapeDtypeStruct((N, 1), jnp.float32),
            jax.ShapeDtypeStruct((N, 1), jnp.float32),
        ],
    )(x, W1, a1_src, a1_dst)

    e1 = _e_pass(as1.reshape(N), ad1.reshape(N), src, dst)
    p1, q1 = _edge_l1(h1, src, dst, e1, zh1, ze)

    h2, as2, ad2 = pl.pallas_call(
        _combine_mm_body,
        out_shape=[
            jax.ShapeDtypeStruct((N, COUT), jnp.float32),
            jax.ShapeDtypeStruct((N, 1), jnp.float32),
            jax.ShapeDtypeStruct((N, 1), jnp.float32),
        ],
    )(p1, q1, b1, W2, a2_src, a2_dst)

    e2 = _e_pass(as2.reshape(N), ad2.reshape(N), src, dst)
    p2, q2 = _edge_l2(h2, src, dst, e2, zh2, ze)

    hf = pl.pallas_call(
        _combine_body,
        out_shape=jax.ShapeDtypeStruct((N, COUT), jnp.float32),
    )(p2, q2, b2)

    g = hf.reshape(N // NPB, NPB * COUT)

    out = pl.pallas_call(
        _fc_body,
        out_shape=jax.ShapeDtypeStruct((N // NPB, OUT_DIM), jnp.float32),
    )(g, Wf1, bf1, Wf2, bf2)

    return out
